# baseline (speedup 1.0000x reference)
import jax
import jax.numpy as jnp
from jax.experimental import pallas as pl

_F_IN = 128
_F_OUT = 32
_BLOCK = 1000


def _fused_body(x_ref, wxz_ref, bz_ref, wxh_ref, bh_ref, wl_ref, bl_ref,
                o_ref):
    wz = wxz_ref[0, 0, :_F_IN, :] + wxz_ref[1, 0, :_F_IN, :]
    wh = wxh_ref[0, 0, :_F_IN, :] + wxh_ref[1, 0, :_F_IN, :]
    xb = x_ref[...]
    z = jax.nn.sigmoid(
        jnp.dot(xb, wz, preferred_element_type=jnp.float32) + bz_ref[...])
    h_tilde = jnp.tanh(
        jnp.dot(xb, wh, preferred_element_type=jnp.float32) + bh_ref[...])
    h = jnp.maximum((1.0 - z) * h_tilde, 0.0)
    o_ref[...] = jnp.dot(h, wl_ref[...],
                         preferred_element_type=jnp.float32) + bl_ref[...]


def kernel(x, edge_index, W_xz, b_z, W_xr, b_r, W_xh, b_h, W_lin, b_lin):
    n = x.shape[0]
    d_cat = W_xz.shape[2]
    grid = (n // _BLOCK,)
    whole = lambda *shape: pl.BlockSpec(shape, lambda i: (0,) * len(shape))
    return pl.pallas_call(
        _fused_body,
        grid=grid,
        in_specs=[
            pl.BlockSpec((_BLOCK, _F_IN), lambda i: (i, 0)),
            whole(2, 1, d_cat, _F_OUT),
            whole(_F_OUT),
            whole(2, 1, d_cat, _F_OUT),
            whole(_F_OUT),
            whole(_F_OUT, 1),
            whole(1),
        ],
        out_specs=pl.BlockSpec((_BLOCK, 1), lambda i: (i, 0)),
        out_shape=jax.ShapeDtypeStruct((n, 1), jnp.float32),
    )(x, W_xz, b_z, W_xh, b_h, W_lin, b_lin)


# speedup vs baseline: 1.2038x; 1.2038x over previous
import jax
import jax.numpy as jnp
from jax.experimental import pallas as pl

_F_IN = 128
_F_OUT = 32
_BLOCK = 5000


def _fused_body(x_ref, wxz_ref, bz_ref, wxh_ref, bh_ref, wl_ref, bl_ref,
                o_ref):
    wz = wxz_ref[0, 0, :_F_IN, :] + wxz_ref[1, 0, :_F_IN, :]
    wh = wxh_ref[0, 0, :_F_IN, :] + wxh_ref[1, 0, :_F_IN, :]
    xb = x_ref[...]
    z = jax.nn.sigmoid(
        jnp.dot(xb, wz, preferred_element_type=jnp.float32) + bz_ref[...])
    h_tilde = jnp.tanh(
        jnp.dot(xb, wh, preferred_element_type=jnp.float32) + bh_ref[...])
    h = jnp.maximum((1.0 - z) * h_tilde, 0.0)
    o_ref[...] = jnp.dot(h, wl_ref[...],
                         preferred_element_type=jnp.float32) + bl_ref[...]


def kernel(x, edge_index, W_xz, b_z, W_xr, b_r, W_xh, b_h, W_lin, b_lin):
    n = x.shape[0]
    d_cat = W_xz.shape[2]
    grid = (n // _BLOCK,)
    whole = lambda *shape: pl.BlockSpec(shape, lambda i: (0,) * len(shape))
    return pl.pallas_call(
        _fused_body,
        grid=grid,
        in_specs=[
            pl.BlockSpec((_BLOCK, _F_IN), lambda i: (i, 0)),
            whole(2, 1, d_cat, _F_OUT),
            whole(_F_OUT),
            whole(2, 1, d_cat, _F_OUT),
            whole(_F_OUT),
            whole(_F_OUT, 1),
            whole(1),
        ],
        out_specs=pl.BlockSpec((_BLOCK, 1), lambda i: (i, 0)),
        out_shape=jax.ShapeDtypeStruct((n, 1), jnp.float32),
    )(x, W_xz, b_z, W_xh, b_h, W_lin, b_lin)


# speedup vs baseline: 1.2392x; 1.0293x over previous
import jax
import jax.numpy as jnp
from jax.experimental import pallas as pl

_F_IN = 128
_F_OUT = 32
_BLOCK = 10000


def _fused_body(x_ref, wxz_ref, bz_ref, wxh_ref, bh_ref, wl_ref, bl_ref,
                o_ref):
    wz = wxz_ref[0, 0, :_F_IN, :] + wxz_ref[1, 0, :_F_IN, :]
    wh = wxh_ref[0, 0, :_F_IN, :] + wxh_ref[1, 0, :_F_IN, :]
    xb = x_ref[...]
    z = jax.nn.sigmoid(
        jnp.dot(xb, wz, preferred_element_type=jnp.float32) + bz_ref[...])
    h_tilde = jnp.tanh(
        jnp.dot(xb, wh, preferred_element_type=jnp.float32) + bh_ref[...])
    h = jnp.maximum((1.0 - z) * h_tilde, 0.0)
    o_ref[...] = jnp.dot(h, wl_ref[...],
                         preferred_element_type=jnp.float32) + bl_ref[...]


def kernel(x, edge_index, W_xz, b_z, W_xr, b_r, W_xh, b_h, W_lin, b_lin):
    n = x.shape[0]
    d_cat = W_xz.shape[2]
    grid = (n // _BLOCK,)
    whole = lambda *shape: pl.BlockSpec(shape, lambda i: (0,) * len(shape))
    return pl.pallas_call(
        _fused_body,
        grid=grid,
        in_specs=[
            pl.BlockSpec((_BLOCK, _F_IN), lambda i: (i, 0)),
            whole(2, 1, d_cat, _F_OUT),
            whole(_F_OUT),
            whole(2, 1, d_cat, _F_OUT),
            whole(_F_OUT),
            whole(_F_OUT, 1),
            whole(1),
        ],
        out_specs=pl.BlockSpec((_BLOCK, 1), lambda i: (i, 0)),
        out_shape=jax.ShapeDtypeStruct((n, 1), jnp.float32),
    )(x, W_xz, b_z, W_xh, b_h, W_lin, b_lin)
